# baseline (speedup 1.0000x reference)
import functools

import jax
import jax.numpy as jnp
from jax import lax
from jax.experimental import pallas as pl
from jax.experimental.pallas import tpu as pltpu
from jax.experimental.pallas import tpu_sc as plsc

B = 16384
D = 64
DP = D // 2
NEGS = 20
NJ = NEGS + 1
NC = 2
NS = 16
NW = NC * NS
BPW = B // NW
CB = 32
NCHUNK = BPW // CB
XROWS = CB * NJ
IDXW = 112


def _sc_body(center_hbm, pn_hbm, ctab_hbm, ntab_hbm, out_hbm,
             cidx, xidx, crow, xrow, scores, sem_c, sem_x):
    wid = lax.axis_index("s") * NC + lax.axis_index("c")
    base = wid * BPW

    def chunk_body(c, carry):
        cb0 = base + c * CB
        pltpu.sync_copy(center_hbm.at[pl.ds(cb0, CB)], cidx)
        pltpu.sync_copy(pn_hbm.at[pl.ds(cb0 * NJ, XROWS)], xidx)

        copies = [pltpu.async_copy(ctab_hbm.at[cidx], crow, sem_c)]
        for r in range(XROWS // IDXW):
            copies.append(pltpu.async_copy(
                ntab_hbm.at[xidx.at[pl.ds(r * IDXW, IDXW)]],
                xrow.at[pl.ds(r * IDXW, IDXW)], sem_x))
        for cp in copies:
            cp.wait()

        def group_body(g, carry2):
            lanes = lax.iota(jnp.int32, 16) + g * 16
            off = c * CB + g * 16
            rot = lax.iota(jnp.int32, 16)
            xlanes = [lanes * NJ + j for j in range(NJ)]

            def mul2(cw, xw):
                ca, cb_ = plsc.unpack(plsc.bitcast(cw, jnp.bfloat16),
                                      format=plsc.PackFormat.INTERLEAVED)
                xa, xb = plsc.unpack(plsc.bitcast(xw, jnp.bfloat16),
                                     format=plsc.PackFormat.INTERLEAVED)
                return ca * xa + cb_ * xb

            def p_block(k, accs):
                pv0 = rot + k * 16
                accs = list(accs)
                for t in range(16):
                    pv = (pv0 + t) & (DP - 1)
                    cw = plsc.load_gather(crow, [lanes, pv])
                    for j in range(NJ):
                        accs[j] = accs[j] + mul2(
                            cw, plsc.load_gather(xrow, [xlanes[j], pv]))
                return tuple(accs)

            zero = jnp.zeros((16,), jnp.float32)
            accs = lax.fori_loop(0, DP // 16, p_block,
                                 tuple(zero for _ in range(NJ)))
            for j in range(NJ):
                scores[pl.ds(j * BPW + off, 16)] = accs[j]
            return carry2

        lax.fori_loop(0, CB // 16, group_body, 0)
        return carry

    lax.fori_loop(0, NCHUNK, chunk_body, 0)
    for j in range(NJ):
        pltpu.sync_copy(scores.at[pl.ds(j * BPW, BPW)],
                        out_hbm.at[pl.ds(j * B + base, BPW)])


_sc_scores = functools.partial(
    pl.kernel,
    out_type=jax.ShapeDtypeStruct((NJ * B,), jnp.float32),
    mesh=plsc.VectorSubcoreMesh(core_axis_name="c", subcore_axis_name="s"),
    compiler_params=pltpu.CompilerParams(
        needs_layout_passes=False, use_tc_tiling_on_sc=False),
    scratch_types=[
        pltpu.VMEM((CB,), jnp.int32),
        pltpu.VMEM((XROWS,), jnp.int32),
        pltpu.VMEM((CB, DP), jnp.float32),
        pltpu.VMEM((XROWS, DP), jnp.float32),
        pltpu.VMEM((NJ * BPW,), jnp.float32),
        pltpu.SemaphoreType.DMA,
        pltpu.SemaphoreType.DMA,
    ],
)(_sc_body)


def _pack_table(tab):
    return jax.lax.bitcast_convert_type(
        tab.astype(jnp.bfloat16).reshape(tab.shape[0], DP, 2), jnp.float32)


def _tc_loss(x_ref, o_ref):
    x = jnp.clip(x_ref[...], -10.0, 10.0)
    ls = -jnp.log1p(jnp.exp(-x))
    o_ref[0, 0] = -jnp.sum(ls) / B


def kernel(center, pos, neg, center_table, neigh_table):
    center = center.astype(jnp.int32)
    pn = jnp.concatenate(
        [pos.astype(jnp.int32)[:, None], neg.astype(jnp.int32)], axis=1)
    pn = pn.reshape(B * NJ)
    scores = _sc_scores(center, pn, _pack_table(center_table),
                        _pack_table(neigh_table))
    flat = scores.reshape(NJ * B // 128, 128)
    loss = pl.pallas_call(
        _tc_loss,
        out_shape=jax.ShapeDtypeStruct((1, 1), jnp.float32),
        out_specs=pl.BlockSpec(memory_space=pltpu.SMEM),
    )(flat)
    return loss[0, 0]


# speedup vs baseline: 2.7238x; 2.7238x over previous
import functools

import jax
import jax.numpy as jnp
from jax import lax
from jax.experimental import pallas as pl
from jax.experimental.pallas import tpu as pltpu
from jax.experimental.pallas import tpu_sc as plsc

B = 16384
D = 64
NEGS = 20
NJ = NEGS + 1
NC = 2
NS = 16
NW = NC * NS
BPW = B // NW
CB = 16
NCHUNK = BPW // CB
XROWS = CB * NJ
IDXW = 112


def _sc_body(center_hbm, pn_hbm, ctab_hbm, ntab_hbm, out_hbm,
             cidx, xidx, crow, xrow, scores,
             semg0, semg1, semi0, semi1):
    wid = lax.axis_index("s") * NC + lax.axis_index("c")
    base = wid * BPW
    semg = (semg0, semg1)
    semi = (semi0, semi1)

    def idx_copies(c, s):
        cb0 = base + c * CB
        return [
            pltpu.make_async_copy(center_hbm.at[pl.ds(cb0, CB)],
                                  cidx.at[pl.ds(s * CB, CB)], semi[s]),
            pltpu.make_async_copy(pn_hbm.at[pl.ds(cb0 * NJ, XROWS)],
                                  xidx.at[pl.ds(s * XROWS, XROWS)], semi[s]),
        ]

    def gather_copies(s):
        cps = [pltpu.make_async_copy(ctab_hbm.at[cidx.at[pl.ds(s * CB, CB)]],
                                     crow.at[s], semg[s])]
        for r in range(XROWS // IDXW):
            cps.append(pltpu.make_async_copy(
                ntab_hbm.at[xidx.at[pl.ds(s * XROWS + r * IDXW, IDXW)]],
                xrow.at[s, pl.ds(r * IDXW, IDXW)], semg[s]))
        return cps

    def fire(cps):
        for cp in cps:
            cp.start()

    def drain(cps):
        for cp in cps:
            cp.wait()

    def compute(c, s):
        lanes = lax.iota(jnp.int32, 16)
        off = c * CB
        rot = lanes
        xlanes = [lanes * NJ + j for j in range(NJ)]
        crow_s = crow.at[s]
        xrow_s = xrow.at[s]

        def d_block(k, accs):
            dv0 = rot + k * 16
            accs = list(accs)
            for t in range(16):
                dv = (dv0 + t) & (D - 1)
                cT = plsc.load_gather(crow_s, [lanes, dv])
                for j in range(NJ):
                    accs[j] = accs[j] + cT * plsc.load_gather(
                        xrow_s, [xlanes[j], dv])
            return tuple(accs)

        zero = jnp.zeros((16,), jnp.float32)
        accs = lax.fori_loop(0, D // 16, d_block,
                             tuple(zero for _ in range(NJ)))
        for j in range(NJ):
            scores[pl.ds(j * BPW + off, 16)] = accs[j]

    c0 = idx_copies(0, 0)
    c1 = idx_copies(1, 1)
    fire(c0)
    fire(c1)
    drain(c0)
    fire(gather_copies(0))
    drain(c1)

    def pair_body(c2, carry):
        a = 2 * c2
        more = a + 2 < NCHUNK

        drain(gather_copies(0))
        fire(gather_copies(1))

        @pl.when(more)
        def _():
            fire(idx_copies(a + 2, 0))
        compute(a, 0)

        drain(gather_copies(1))

        @pl.when(more)
        def _():
            fire(idx_copies(a + 3, 1))
            drain(idx_copies(a + 2, 0))
            fire(gather_copies(0))
        compute(a + 1, 1)

        @pl.when(more)
        def _():
            drain(idx_copies(a + 3, 1))
        return carry

    lax.fori_loop(0, NCHUNK // 2, pair_body, 0)

    for j in range(NJ):
        pltpu.sync_copy(scores.at[pl.ds(j * BPW, BPW)],
                        out_hbm.at[pl.ds(j * B + base, BPW)])


_sc_scores = functools.partial(
    pl.kernel,
    out_type=jax.ShapeDtypeStruct((NJ * B,), jnp.float32),
    mesh=plsc.VectorSubcoreMesh(core_axis_name="c", subcore_axis_name="s"),
    compiler_params=pltpu.CompilerParams(
        needs_layout_passes=False, use_tc_tiling_on_sc=False),
    scratch_types=[
        pltpu.VMEM((2 * CB,), jnp.int32),
        pltpu.VMEM((2 * XROWS,), jnp.int32),
        pltpu.VMEM((2, CB, D), jnp.float32),
        pltpu.VMEM((2, XROWS, D), jnp.float32),
        pltpu.VMEM((NJ * BPW,), jnp.float32),
        pltpu.SemaphoreType.DMA,
        pltpu.SemaphoreType.DMA,
        pltpu.SemaphoreType.DMA,
        pltpu.SemaphoreType.DMA,
    ],
)(_sc_body)


def _tc_loss(x_ref, o_ref):
    x = jnp.clip(x_ref[...], -10.0, 10.0)
    ls = -jnp.log1p(jnp.exp(-x))
    o_ref[0, 0] = -jnp.sum(ls) / B


def kernel(center, pos, neg, center_table, neigh_table):
    center = center.astype(jnp.int32)
    pn = jnp.concatenate(
        [pos.astype(jnp.int32)[:, None], neg.astype(jnp.int32)], axis=1)
    pn = pn.reshape(B * NJ)
    scores = _sc_scores(center, pn, center_table, neigh_table)
    flat = scores.reshape(NJ * B // 128, 128)
    loss = pl.pallas_call(
        _tc_loss,
        out_shape=jax.ShapeDtypeStruct((1, 1), jnp.float32),
        out_specs=pl.BlockSpec(memory_space=pltpu.SMEM),
    )(flat)
    return loss[0, 0]
